# Initial kernel scaffold; baseline (speedup 1.0000x reference)
import jax
import jax.numpy as jnp
from jax.experimental import pallas as pl


def kernel(bx, by_buf, ents_buf, logits_buf, x, y, ents, logits, idx):
    raise NotImplementedError("write your pallas kernel here")


# speedup vs baseline: 1.3673x; 1.3673x over previous
import functools

import jax
import jax.numpy as jnp
from jax import lax
from jax.experimental import pallas as pl
from jax.experimental.pallas import tpu as pltpu
from jax.experimental.pallas import tpu_sc as plsc

M, D, B, C = 100000, 256, 16384, 100
CP = 128
NC, NS = 2, 16
NW = NC * NS
RB = 160
NBLK = M // RB
TPW = (NBLK + NW - 1) // NW
L = 16
NG = RB // L

_mesh = plsc.VectorSubcoreMesh(core_axis_name="c", subcore_axis_name="s",
                               num_cores=NC, num_subcores=NS)


@functools.partial(
    pl.kernel,
    out_type=(
        jax.ShapeDtypeStruct((M,), jnp.int32),
        jax.ShapeDtypeStruct((M,), jnp.float32),
        jax.ShapeDtypeStruct((M, C), jnp.float32),
    ),
    mesh=_mesh,
    compiler_params=pltpu.CompilerParams(needs_layout_passes=False),
    scratch_types=[
        pltpu.VMEM((RB, C), jnp.float32),
        pltpu.VMEM((RB,), jnp.int32),
        pltpu.VMEM((RB,), jnp.float32),
        pltpu.VMEM((RB,), jnp.int32),
        pltpu.VMEM((RB + L,), jnp.int32),
        pltpu.VMEM((RB + L,), jnp.int32),
        pltpu.VMEM((L, D), jnp.float32),
        pltpu.VMEM((L, CP), jnp.float32),
        pltpu.VMEM((L,), jnp.int32),
        pltpu.VMEM((L,), jnp.float32),
        pltpu.VMEM((L,), jnp.int32),
        pltpu.SemaphoreType.DMA,
        pltpu.SemaphoreType.DMA,
        pltpu.SemaphoreType.DMA,
    ],
)
def _patch(bx_ref, by_in, ents_in, logits_in, x, y, ents, logits_pad, owner,
           out_by, out_ents, out_logits,
           lgb, byb, entsb, ownb, tgt_l, src_l, xs, ls, ys, es, tabs,
           bsem, gsem, ssem):
    w = lax.axis_index("s") * NC + lax.axis_index("c")
    iota = lax.broadcasted_iota(jnp.int32, (L,), 0)

    def do_block(bk):
        r0 = bk * RB
        cb = pltpu.make_async_copy(logits_in.at[pl.ds(r0, RB)], lgb, bsem)
        cb.start()
        cy = pltpu.make_async_copy(by_in.at[pl.ds(r0, RB)], byb, bsem)
        cy.start()
        ce = pltpu.make_async_copy(ents_in.at[pl.ds(r0, RB)], entsb, bsem)
        ce.start()
        pltpu.sync_copy(owner.at[pl.ds(r0, RB)], ownb)

        cnt = jnp.int32(0)
        maxpack = jnp.int32(-1)
        for g in range(NG):
            ov = ownb[pl.ds(g * L, L)]
            mask = ov >= 0
            pos = iota + (g * L)
            cpos = plsc.cumsum(mask.astype(jnp.int32)) - 1 + cnt
            plsc.store_scatter(tgt_l, [cpos], pos, mask=mask)
            plsc.store_scatter(src_l, [cpos], ov, mask=mask)
            pack = jnp.where(mask, pos * 32768 + ov, -1)
            maxpack = jnp.maximum(maxpack, jnp.max(pack))
            cnt = cnt + jnp.sum(mask.astype(jnp.int32))

        @pl.when(cnt > 0)
        def _():
            pt = maxpack >> 15
            ps = maxpack & 32767
            tgt_l[pl.ds(cnt, L)] = jnp.full((L,), 1, jnp.int32) * pt
            src_l[pl.ds(cnt, L)] = jnp.full((L,), 1, jnp.int32) * ps

        cb.wait()
        cy.wait()
        ce.wait()

        ngrp = (cnt + (L - 1)) // L

        def grp(g, carry):
            tl = tgt_l[pl.ds(g * L, L)]
            sl = src_l[pl.ds(g * L, L)]
            pltpu.async_copy(x.at[sl], xs, gsem).wait()
            tabs[...] = tl + r0
            pltpu.async_copy(xs, bx_ref.at[tabs], ssem).wait()
            pltpu.async_copy(logits_pad.at[sl], ls, gsem).wait()
            for c in range(C):
                col = jnp.full((L,), c, jnp.int32)
                v = plsc.load_gather(ls, [iota, col])
                plsc.store_scatter(lgb, [tl, col], v)
            pltpu.async_copy(y.at[sl], ys, gsem).wait()
            plsc.store_scatter(byb, [tl], ys[...])
            pltpu.async_copy(ents.at[sl], es, gsem).wait()
            plsc.store_scatter(entsb, [tl], es[...])
            return carry

        lax.fori_loop(0, ngrp, grp, 0)

        pltpu.async_copy(lgb, out_logits.at[pl.ds(r0, RB)], bsem).wait()
        pltpu.async_copy(byb, out_by.at[pl.ds(r0, RB)], bsem).wait()
        pltpu.async_copy(entsb, out_ents.at[pl.ds(r0, RB)], bsem).wait()

    def slot(t, carry):
        bk = w + NW * t

        @pl.when(bk < NBLK)
        def _():
            do_block(bk)

        return carry

    lax.fori_loop(0, TPW, slot, 0)


def kernel(bx, by_buf, ents_buf, logits_buf, x, y, ents, logits, idx):
    js = jnp.arange(B, dtype=jnp.int32)
    owner = jnp.full((M,), -1, jnp.int32).at[idx].set(js)
    logits_pad = jnp.pad(logits, ((0, 0), (0, CP - C)))
    rbx = jax.new_ref(bx)
    out_by, out_ents, out_logits = _patch(
        rbx, by_buf, ents_buf, logits_buf, x, y, ents, logits_pad, owner)
    return jax.freeze(rbx), out_by, out_ents, out_logits


# speedup vs baseline: 1.8127x; 1.3258x over previous
import functools

import jax
import jax.numpy as jnp
from jax import lax
from jax.experimental import pallas as pl
from jax.experimental.pallas import tpu as pltpu
from jax.experimental.pallas import tpu_sc as plsc

M, D, B, C = 100000, 256, 16384, 100
CP = 128
NC, NS = 2, 16
NW = NC * NS
RB = 80
NBLK = M // RB
TPW = (NBLK + NW - 1) // NW
L = 16
NG = RB // L
NGF = 3
NSEC = TPW + 4
NIT = (NSEC + 3) // 4

_mesh = plsc.VectorSubcoreMesh(core_axis_name="c", subcore_axis_name="s",
                               num_cores=NC, num_subcores=NS)


@functools.partial(
    pl.kernel,
    out_type=(
        jax.ShapeDtypeStruct((M,), jnp.int32),
        jax.ShapeDtypeStruct((M,), jnp.float32),
        jax.ShapeDtypeStruct((M, C), jnp.float32),
    ),
    mesh=_mesh,
    compiler_params=pltpu.CompilerParams(needs_layout_passes=False),
    scratch_types=[
        pltpu.VMEM((4, RB, C), jnp.float32),
        pltpu.VMEM((4, RB), jnp.int32),
        pltpu.VMEM((4, RB), jnp.float32),
        pltpu.VMEM((4, RB), jnp.int32),
        pltpu.VMEM((2, RB + L), jnp.int32),
        pltpu.VMEM((2, RB + L), jnp.int32),
        pltpu.VMEM((2, NGF, L, D), jnp.float32),
        pltpu.VMEM((2, NGF, L, CP), jnp.float32),
        pltpu.VMEM((L,), jnp.int32),
        pltpu.VMEM((L,), jnp.int32),
        pltpu.VMEM((L,), jnp.int32),
        pltpu.VMEM((L,), jnp.int32),
        pltpu.VMEM((L,), jnp.int32),
        pltpu.VMEM((L,), jnp.int32),
        pltpu.VMEM((L,), jnp.float32),
        pltpu.VMEM((L,), jnp.float32),
        pltpu.VMEM((L,), jnp.float32),
        pltpu.VMEM((L,), jnp.float32),
        pltpu.VMEM((L,), jnp.float32),
        pltpu.VMEM((L,), jnp.float32),
        pltpu.VMEM((L,), jnp.int32),
        pltpu.VMEM((L,), jnp.int32),
        pltpu.VMEM((L,), jnp.int32),
        pltpu.VMEM((L,), jnp.int32),
        pltpu.VMEM((L,), jnp.int32),
        pltpu.VMEM((L,), jnp.int32),
        pltpu.VMEM((L, D), jnp.float32),
        pltpu.VMEM((L, CP), jnp.float32),
        pltpu.VMEM((L,), jnp.int32),
        pltpu.VMEM((L,), jnp.float32),
        pltpu.VMEM((L,), jnp.int32),
        pltpu.SMEM((8,), jnp.int32),
        pltpu.SemaphoreType.DMA,
        pltpu.SemaphoreType.DMA,
        pltpu.SemaphoreType.DMA,
        pltpu.SemaphoreType.DMA,
        pltpu.SemaphoreType.DMA,
        pltpu.SemaphoreType.DMA,
        pltpu.SemaphoreType.DMA,
        pltpu.SemaphoreType.DMA,
        pltpu.SemaphoreType.DMA,
        pltpu.SemaphoreType.DMA,
        pltpu.SemaphoreType.DMA,
        pltpu.SemaphoreType.DMA,
    ],
)
def _patch(bx_ref, by_in, ents_in, logits_in, x, y, ents, logits_pad, owner,
           out_by, out_ents, out_logits,
           lgb, byb, entsb, ownb, tgt_l, src_l, xs, ls,
           ys00, ys01, ys02, ys10, ys11, ys12,
           es00, es01, es02, es10, es11, es12,
           tb00, tb01, tb02, tb10, tb11, tb12,
           sxs, sls, sys_, ses, stabs, cnts,
           bsem0, bsem1, bsem2, bsem3, wsem0, wsem1, wsem2, wsem3,
           gsem0, gsem1, ssem, slsem):
    w = lax.axis_index("s") * NC + lax.axis_index("c")
    iota = lax.broadcasted_iota(jnp.int32, (L,), 0)
    ys = ((ys00, ys01, ys02), (ys10, ys11, ys12))
    es = ((es00, es01, es02), (es10, es11, es12))
    tabs = ((tb00, tb01, tb02), (tb10, tb11, tb12))
    bsem = (bsem0, bsem1, bsem2, bsem3)
    wsem = (wsem0, wsem1, wsem2, wsem3)
    gsem = (gsem0, gsem1)

    def in_copies(t, a):
        r0 = (w + NW * t) * RB
        return (
            pltpu.make_async_copy(logits_in.at[pl.ds(r0, RB)], lgb.at[a],
                                  bsem[a]),
            pltpu.make_async_copy(by_in.at[pl.ds(r0, RB)], byb.at[a],
                                  bsem[a]),
            pltpu.make_async_copy(ents_in.at[pl.ds(r0, RB)], entsb.at[a],
                                  bsem[a]),
            pltpu.make_async_copy(owner.at[pl.ds(r0, RB)], ownb.at[a],
                                  bsem[a]),
        )

    def out_copies(t, a):
        r0 = (w + NW * t) * RB
        return (
            pltpu.make_async_copy(lgb.at[a], out_logits.at[pl.ds(r0, RB)],
                                  wsem[a]),
            pltpu.make_async_copy(byb.at[a], out_by.at[pl.ds(r0, RB)],
                                  wsem[a]),
            pltpu.make_async_copy(entsb.at[a], out_ents.at[pl.ds(r0, RB)],
                                  wsem[a]),
        )

    def gather_group(t, p, g):
        sl = src_l[p, pl.ds(g * L, L)]
        return (
            pltpu.make_async_copy(x.at[sl], xs.at[p].at[g], gsem[p]),
            pltpu.make_async_copy(logits_pad.at[sl], ls.at[p].at[g],
                                  gsem[p]),
            pltpu.make_async_copy(y.at[sl], ys[p][g], gsem[p]),
            pltpu.make_async_copy(ents.at[sl], es[p][g], gsem[p]),
        )

    def front(t, a):
        p = a % 2
        bk = w + NW * t

        @pl.when(bk < NBLK)
        def _():
            for cpy in in_copies(t, a):
                cpy.wait()
            cnt = jnp.int32(0)
            maxpack = jnp.int32(-1)
            for g in range(NG):
                ov = ownb[a, pl.ds(g * L, L)]
                mask = ov >= 0
                pos = iota + (g * L)
                cpos = plsc.cumsum(mask.astype(jnp.int32)) - 1 + cnt
                plsc.store_scatter(tgt_l.at[p], [cpos], pos, mask=mask)
                plsc.store_scatter(src_l.at[p], [cpos], ov, mask=mask)
                pack = jnp.where(mask, pos * 32768 + ov, -1)
                maxpack = jnp.maximum(maxpack, jnp.max(pack))
                cnt = cnt + jnp.sum(mask.astype(jnp.int32))

            @pl.when(cnt > 0)
            def _():
                pt = maxpack >> 15
                ps = maxpack & 32767
                tgt_l[p, pl.ds(cnt, L)] = jnp.full((L,), 1, jnp.int32) * pt
                src_l[p, pl.ds(cnt, L)] = jnp.full((L,), 1, jnp.int32) * ps

            cnts[p] = cnt
            ngrp = (cnt + (L - 1)) // L
            for g in range(NGF):
                @pl.when(g < ngrp)
                def _(g=g):
                    for cpy in gather_group(t, p, g):
                        cpy.start()

    def back(t, a):
        p = a % 2
        bk = w + NW * t

        @pl.when(bk < NBLK)
        def _():
            r0 = bk * RB
            cnt = cnts[p]
            ngrp = (cnt + (L - 1)) // L
            nfast = jnp.minimum(ngrp, NGF)
            for g in range(NGF):
                @pl.when(g < ngrp)
                def _(g=g):
                    for cpy in gather_group(t, p, g):
                        cpy.wait()
                    tl = tgt_l[p, pl.ds(g * L, L)]
                    tabs[p][g][...] = tl + r0
                    pltpu.make_async_copy(xs.at[p].at[g],
                                          bx_ref.at[tabs[p][g]],
                                          ssem).start()
                    for c in range(C):
                        col = jnp.full((L,), c, jnp.int32)
                        v = plsc.load_gather(ls.at[p].at[g], [iota, col])
                        plsc.store_scatter(lgb.at[a], [tl, col], v)
                    plsc.store_scatter(byb.at[a], [tl], ys[p][g][...])
                    plsc.store_scatter(entsb.at[a], [tl], es[p][g][...])

            def sdrain(g, c):
                pltpu.make_async_copy(xs.at[p].at[0],
                                      bx_ref.at[tabs[p][0]],
                                      ssem).wait()
                return c

            lax.fori_loop(0, nfast, sdrain, 0)

            def slow(g, c):
                sl = src_l[p, pl.ds(g * L, L)]
                pltpu.make_async_copy(x.at[sl], sxs, slsem).start()
                pltpu.make_async_copy(logits_pad.at[sl], sls, slsem).start()
                pltpu.make_async_copy(y.at[sl], sys_, slsem).start()
                pltpu.make_async_copy(ents.at[sl], ses, slsem).start()
                pltpu.make_async_copy(x.at[sl], sxs, slsem).wait()
                pltpu.make_async_copy(logits_pad.at[sl], sls, slsem).wait()
                pltpu.make_async_copy(y.at[sl], sys_, slsem).wait()
                pltpu.make_async_copy(ents.at[sl], ses, slsem).wait()
                tl = tgt_l[p, pl.ds(g * L, L)]
                stabs[pl.ds(0, L)] = tl + r0
                pltpu.make_async_copy(sxs, bx_ref.at[stabs], slsem).start()
                for c2 in range(C):
                    col = jnp.full((L,), c2, jnp.int32)
                    v = plsc.load_gather(sls, [iota, col])
                    plsc.store_scatter(lgb.at[a], [tl, col], v)
                plsc.store_scatter(byb.at[a], [tl], sys_[...])
                plsc.store_scatter(entsb.at[a], [tl], ses[...])
                pltpu.make_async_copy(sxs, bx_ref.at[stabs], slsem).wait()
                return c

            lax.fori_loop(NGF, ngrp, slow, 0)

            for cpy in out_copies(t, a):
                cpy.start()

    def section(t, a):
        bk_m2 = w + NW * (t - 2)

        @pl.when((t >= 2) & (bk_m2 < NBLK))
        def _():
            for cpy in out_copies(t - 2, (a + 2) % 4):
                cpy.wait()

        bk_p1 = w + NW * (t + 1)

        @pl.when(bk_p1 < NBLK)
        def _():
            for cpy in in_copies(t + 1, (a + 1) % 4):
                cpy.start()

        front(t, a)

        @pl.when(t >= 1)
        def _():
            back(t - 1, (a + 3) % 4)

    for cpy in in_copies(0, 0):
        cpy.start()

    def body(i, c):
        t0 = i * 4
        section(t0, 0)
        section(t0 + 1, 1)
        section(t0 + 2, 2)
        section(t0 + 3, 3)
        return c

    lax.fori_loop(0, NIT, body, 0)


def kernel(bx, by_buf, ents_buf, logits_buf, x, y, ents, logits, idx):
    js = jnp.arange(B, dtype=jnp.int32)
    owner = jnp.full((M,), -1, jnp.int32).at[idx].set(js)
    logits_pad = jnp.pad(logits, ((0, 0), (0, CP - C)))
    rbx = jax.new_ref(bx)
    out_by, out_ents, out_logits = _patch(
        rbx, by_buf, ents_buf, logits_buf, x, y, ents, logits_pad, owner)
    return jax.freeze(rbx), out_by, out_ents, out_logits
